# Initial kernel scaffold; baseline (speedup 1.0000x reference)
import jax
import jax.numpy as jnp
from jax.experimental import pallas as pl


def kernel(x, adj, W1, b1, Wd, bd):
    raise NotImplementedError("write your pallas kernel here")


# speedup vs baseline: 6.5850x; 6.5850x over previous
import functools

import jax
import jax.numpy as jnp
from jax import lax
from jax.experimental import pallas as pl
from jax.experimental.pallas import tpu as pltpu
from jax.experimental.pallas import tpu_sc as plsc

_N = 10000
_E = 320000
_D = 128
_H = 128
_C = 64

_NC = 2
_NS = 16
_NW = _NC * _NS
_EPW = _E // _NW
_CHUNK = 125
_NCH = _EPW // _CHUNK

_NP = 10240
_RPT = _NP // _NS
_OCH = 80
_NOCH = _RPT // _OCH

_BR = 80
_GB = 125
_PB = _NP // _BR


def _mm1_body(x_ref, w_ref, o_ref):
    o_ref[...] = jnp.dot(x_ref[...], w_ref[...],
                         preferred_element_type=jnp.float32)


def _head_body(pa_ref, pb_ref, b1_ref, wd_ref, bd_ref, o_ref):
    h = jax.nn.relu(pa_ref[...] + pb_ref[...] + b1_ref[...])
    o_ref[...] = jnp.dot(h, wd_ref[...],
                         preferred_element_type=jnp.float32) + bd_ref[...]


_sc_mesh = plsc.VectorSubcoreMesh(core_axis_name="c", subcore_axis_name="s")


@functools.partial(
    pl.kernel,
    mesh=_sc_mesh,
    out_type=jax.ShapeDtypeStruct((_NC * _NP, _H), jnp.float32),
    scratch_types=[
        pltpu.VMEM((_NCH, _CHUNK), jnp.int32),
        pltpu.VMEM((_NCH, _CHUNK), jnp.int32),
        pltpu.VMEM((_CHUNK, _H), jnp.float32),
        pltpu.VMEM_SHARED((_NP, _H), jnp.float32),
        pltpu.SemaphoreType.DMA,
    ],
)
def _sc_agg(support_hbm, src_hbm, dst_hbm, out_hbm,
            src_v, dst_v, rows_v, acc_sh, sem):
    cid = lax.axis_index("c")
    sid = lax.axis_index("s")
    wid = sid * _NC + cid

    def _zero_row(r, carry):
        for c in range(_H // 16):
            rows_v[r, pl.ds(c * 16, 16)] = jnp.zeros((16,), jnp.float32)
        return carry
    lax.fori_loop(0, _OCH, _zero_row, 0)
    for t in range(_NOCH):
        pltpu.sync_copy(rows_v.at[pl.ds(0, _OCH)],
                        acc_sh.at[pl.ds(sid * _RPT + t * _OCH, _OCH)])
    plsc.subcore_barrier()

    pltpu.sync_copy(src_hbm.at[wid], src_v)
    pltpu.sync_copy(dst_hbm.at[wid], dst_v)

    def _edge_chunk(j, carry):
        pltpu.async_copy(support_hbm.at[src_v.at[j]], rows_v, sem).wait()
        pltpu.sync_copy(rows_v, acc_sh.at[dst_v.at[j]], add=True)
        return carry
    lax.fori_loop(0, _NCH, _edge_chunk, 0)
    plsc.subcore_barrier()

    for t in range(_NOCH):
        base = sid * _RPT + t * _OCH
        pltpu.sync_copy(acc_sh.at[pl.ds(base, _OCH)],
                        rows_v.at[pl.ds(0, _OCH)])
        pltpu.sync_copy(rows_v.at[pl.ds(0, _OCH)],
                        out_hbm.at[pl.ds(cid * _NP + base, _OCH)])


def kernel(x, adj, W1, b1, Wd, bd):
    src3 = adj[0].reshape(_NW, _NCH, _CHUNK)
    dst3 = adj[1].reshape(_NW, _NCH, _CHUNK)

    support = pl.pallas_call(
        _mm1_body,
        grid=(10,),
        in_specs=[pl.BlockSpec((_N // 10, _D), lambda i: (i, 0)),
                  pl.BlockSpec((_D, _H), lambda i: (0, 0))],
        out_specs=pl.BlockSpec((_N // 10, _H), lambda i: (i, 0)),
        out_shape=jax.ShapeDtypeStruct((_N, _H), jnp.float32),
    )(x, W1)

    partials = _sc_agg(support, src3, dst3)

    out = pl.pallas_call(
        _head_body,
        grid=(_GB,),
        in_specs=[pl.BlockSpec((_BR, _H), lambda i: (i, 0)),
                  pl.BlockSpec((_BR, _H), lambda i: (i + _PB, 0)),
                  pl.BlockSpec((1, _H), lambda i: (0, 0)),
                  pl.BlockSpec((_H, _C), lambda i: (0, 0)),
                  pl.BlockSpec((1, _C), lambda i: (0, 0))],
        out_specs=pl.BlockSpec((_BR, _C), lambda i: (i, 0)),
        out_shape=jax.ShapeDtypeStruct((_N, _C), jnp.float32),
    )(partials, partials, b1[None], Wd, bd[None])
    return out


# speedup vs baseline: 7.7787x; 1.1813x over previous
import functools

import jax
import jax.numpy as jnp
from jax import lax
from jax.experimental import pallas as pl
from jax.experimental.pallas import tpu as pltpu
from jax.experimental.pallas import tpu_sc as plsc

_N = 10000
_E = 320000
_D = 128
_H = 128
_C = 64

_NC = 2
_NS = 16
_NW = _NC * _NS
_EPW = _E // _NW
_CHUNK = 125
_NCH = _EPW // _CHUNK

_NP = 10240
_RPT = _NP // _NS
_OCH = 80
_NOCH = _RPT // _OCH

_BR = 80
_GB = 125
_PB = _NP // _BR


def _mm1_body(x_ref, w_ref, o_ref):
    o_ref[...] = jnp.dot(x_ref[...], w_ref[...],
                         preferred_element_type=jnp.float32)


def _head_body(pa_ref, pb_ref, b1_ref, wd_ref, bd_ref, o_ref):
    h = jax.nn.relu(pa_ref[...] + pb_ref[...] + b1_ref[...])
    o_ref[...] = jnp.dot(h, wd_ref[...],
                         preferred_element_type=jnp.float32) + bd_ref[...]


_sc_mesh = plsc.VectorSubcoreMesh(core_axis_name="c", subcore_axis_name="s")


@functools.partial(
    pl.kernel,
    mesh=_sc_mesh,
    out_type=jax.ShapeDtypeStruct((_NC * _NP, _H), jnp.float32),
    scratch_types=[
        pltpu.VMEM((_NCH // 2, _CHUNK), jnp.int32),
        pltpu.VMEM((_NCH // 2, _CHUNK), jnp.int32),
        pltpu.VMEM((2, _CHUNK, _H), jnp.float32),
        pltpu.VMEM_SHARED((_NP, _H), jnp.float32),
        pltpu.SemaphoreType.DMA,
    ],
)
def _sc_agg(support_hbm, src_hbm, dst_hbm, out_hbm,
            src_v, dst_v, rows_v, acc_sh, sem):
    cid = lax.axis_index("c")
    sid = lax.axis_index("s")
    wid = sid * _NC + cid

    def _zero_row(r, carry):
        for c in range(_H // 16):
            rows_v[0, r, pl.ds(c * 16, 16)] = jnp.zeros((16,), jnp.float32)
        return carry
    lax.fori_loop(0, _OCH, _zero_row, 0)
    for t in range(_NOCH):
        pltpu.sync_copy(rows_v.at[0, pl.ds(0, _OCH)],
                        acc_sh.at[pl.ds(sid * _RPT + t * _OCH, _OCH)])
    plsc.subcore_barrier()

    _HCH = _NCH // 2
    for phase in range(2):
        pltpu.sync_copy(src_hbm.at[wid, pl.ds(phase * _HCH, _HCH)], src_v)
        pltpu.sync_copy(dst_hbm.at[wid, pl.ds(phase * _HCH, _HCH)], dst_v)
        pltpu.async_copy(support_hbm.at[src_v.at[0]], rows_v.at[0], sem)

        def _edge_chunk(j, carry):
            b = lax.rem(j, 2)
            pltpu.make_async_copy(support_hbm.at[src_v.at[j]],
                                  rows_v.at[b], sem).wait()
            pltpu.async_copy(support_hbm.at[src_v.at[j + 1]],
                             rows_v.at[1 - b], sem)
            pltpu.sync_copy(rows_v.at[b], acc_sh.at[dst_v.at[j]], add=True)
            return carry
        lax.fori_loop(0, _HCH - 1, _edge_chunk, 0)
        _last = (_HCH - 1) % 2
        pltpu.make_async_copy(support_hbm.at[src_v.at[_HCH - 1]],
                              rows_v.at[_last], sem).wait()
        pltpu.sync_copy(rows_v.at[_last], acc_sh.at[dst_v.at[_HCH - 1]],
                        add=True)
    plsc.subcore_barrier()

    for t in range(_NOCH):
        base = sid * _RPT + t * _OCH
        pltpu.sync_copy(acc_sh.at[pl.ds(base, _OCH)],
                        rows_v.at[0, pl.ds(0, _OCH)])
        pltpu.sync_copy(rows_v.at[0, pl.ds(0, _OCH)],
                        out_hbm.at[pl.ds(cid * _NP + base, _OCH)])


def kernel(x, adj, W1, b1, Wd, bd):
    src3 = adj[0].reshape(_NW, _NCH, _CHUNK)
    dst3 = adj[1].reshape(_NW, _NCH, _CHUNK)

    support = pl.pallas_call(
        _mm1_body,
        grid=(10,),
        in_specs=[pl.BlockSpec((_N // 10, _D), lambda i: (i, 0)),
                  pl.BlockSpec((_D, _H), lambda i: (0, 0))],
        out_specs=pl.BlockSpec((_N // 10, _H), lambda i: (i, 0)),
        out_shape=jax.ShapeDtypeStruct((_N, _H), jnp.float32),
    )(x, W1)

    partials = _sc_agg(support, src3, dst3)

    out = pl.pallas_call(
        _head_body,
        grid=(_GB,),
        in_specs=[pl.BlockSpec((_BR, _H), lambda i: (i, 0)),
                  pl.BlockSpec((_BR, _H), lambda i: (i + _PB, 0)),
                  pl.BlockSpec((1, _H), lambda i: (0, 0)),
                  pl.BlockSpec((_H, _C), lambda i: (0, 0)),
                  pl.BlockSpec((1, _C), lambda i: (0, 0))],
        out_specs=pl.BlockSpec((_BR, _C), lambda i: (i, 0)),
        out_shape=jax.ShapeDtypeStruct((_N, _C), jnp.float32),
    )(partials, partials, b1[None], Wd, bd[None])
    return out


# speedup vs baseline: 10.8462x; 1.3943x over previous
import functools

import jax
import jax.numpy as jnp
from jax import lax
from jax.experimental import pallas as pl
from jax.experimental.pallas import tpu as pltpu
from jax.experimental.pallas import tpu_sc as plsc

_N = 10000
_E = 320000
_D = 128
_H = 128
_C = 64

_NC = 2
_NS = 16
_NW = _NC * _NS
_EPW = _E // _NW
_CHUNK = 125
_NCH = _EPW // _CHUNK
_HCH = _NCH // 2

_NP = 10240
_RPT = _NP // _NS
_OCH = 80
_NOCH = _RPT // _OCH

_BR = 1000
_GB = _N // _BR


def _mm1_body(x_ref, w_ref, o_ref):
    o_ref[...] = jnp.dot(x_ref[...], w_ref[...],
                         preferred_element_type=jnp.float32)


def _head_body(pa_ref, pb_ref, b1_ref, wd_ref, bd_ref, o_ref):
    h = jax.nn.relu(pa_ref[0] + pb_ref[0] + b1_ref[...])
    o_ref[...] = jnp.dot(h, wd_ref[...],
                         preferred_element_type=jnp.float32) + bd_ref[...]


_sc_mesh = plsc.VectorSubcoreMesh(core_axis_name="c", subcore_axis_name="s")


@functools.partial(
    pl.kernel,
    mesh=_sc_mesh,
    out_type=jax.ShapeDtypeStruct((_NC, _NP, _H), jnp.float32),
    scratch_types=[
        pltpu.VMEM((_HCH, _CHUNK), jnp.int32),
        pltpu.VMEM((_HCH, _CHUNK), jnp.int32),
        pltpu.VMEM((2, _CHUNK, _H), jnp.float32),
        pltpu.VMEM_SHARED((_NP, _H), jnp.float32),
        pltpu.SemaphoreType.DMA,
        pltpu.SemaphoreType.DMA,
    ],
)
def _sc_agg(support_hbm, edges_hbm, out_hbm,
            src_v, dst_v, rows_v, acc_sh, sem, osem):
    cid = lax.axis_index("c")
    sid = lax.axis_index("s")
    wid = sid * _NC + cid

    def _zero_row(r, carry):
        for c in range(_H // 16):
            rows_v[0, r, pl.ds(c * 16, 16)] = jnp.zeros((16,), jnp.float32)
        return carry
    lax.fori_loop(0, _OCH, _zero_row, 0)
    for t in range(_NOCH):
        pltpu.sync_copy(rows_v.at[0, pl.ds(0, _OCH)],
                        acc_sh.at[pl.ds(sid * _RPT + t * _OCH, _OCH)])
    plsc.subcore_barrier()

    for phase in range(2):
        pltpu.sync_copy(edges_hbm.at[0, wid, pl.ds(phase * _HCH, _HCH)],
                        src_v)
        pltpu.sync_copy(edges_hbm.at[1, wid, pl.ds(phase * _HCH, _HCH)],
                        dst_v)
        pltpu.async_copy(support_hbm.at[src_v.at[0]], rows_v.at[0], sem)

        def _edge_chunk(j, carry):
            b = lax.rem(j, 2)
            pltpu.make_async_copy(support_hbm.at[src_v.at[j]],
                                  rows_v.at[b], sem).wait()
            pltpu.async_copy(support_hbm.at[src_v.at[j + 1]],
                             rows_v.at[1 - b], sem)
            pltpu.sync_copy(rows_v.at[b], acc_sh.at[dst_v.at[j]], add=True)
            return carry
        lax.fori_loop(0, _HCH - 1, _edge_chunk, 0)
        _last = (_HCH - 1) % 2
        pltpu.make_async_copy(support_hbm.at[src_v.at[_HCH - 1]],
                              rows_v.at[_last], sem).wait()
        pltpu.sync_copy(rows_v.at[_last], acc_sh.at[dst_v.at[_HCH - 1]],
                        add=True)
    plsc.subcore_barrier()

    for t in range(_NOCH):
        base = sid * _RPT + t * _OCH
        pltpu.async_copy(acc_sh.at[pl.ds(base, _OCH)],
                         out_hbm.at[cid, pl.ds(base, _OCH)], osem)
    for t in range(_NOCH):
        pltpu.make_async_copy(acc_sh.at[pl.ds(sid * _RPT, _OCH)],
                              out_hbm.at[cid, pl.ds(sid * _RPT, _OCH)],
                              osem).wait()


def kernel(x, adj, W1, b1, Wd, bd):
    edges = adj.reshape(2, _NW, _NCH, _CHUNK)

    support = pl.pallas_call(
        _mm1_body,
        grid=(10,),
        in_specs=[pl.BlockSpec((_N // 10, _D), lambda i: (i, 0)),
                  pl.BlockSpec((_D, _H), lambda i: (0, 0))],
        out_specs=pl.BlockSpec((_N // 10, _H), lambda i: (i, 0)),
        out_shape=jax.ShapeDtypeStruct((_N, _H), jnp.float32),
    )(x, W1)

    partials = _sc_agg(support, edges)

    out = pl.pallas_call(
        _head_body,
        grid=(_GB,),
        in_specs=[pl.BlockSpec((1, _BR, _H), lambda i: (0, i, 0)),
                  pl.BlockSpec((1, _BR, _H), lambda i: (1, i, 0)),
                  pl.BlockSpec((1, _H), lambda i: (0, 0)),
                  pl.BlockSpec((_H, _C), lambda i: (0, 0)),
                  pl.BlockSpec((1, _C), lambda i: (0, 0))],
        out_specs=pl.BlockSpec((_BR, _C), lambda i: (i, 0)),
        out_shape=jax.ShapeDtypeStruct((_N, _C), jnp.float32),
    )(partials, partials, b1[None], Wd, bd[None])
    return out


# speedup vs baseline: 11.3866x; 1.0498x over previous
import functools

import jax
import jax.numpy as jnp
from jax import lax
from jax.experimental import pallas as pl
from jax.experimental.pallas import tpu as pltpu
from jax.experimental.pallas import tpu_sc as plsc

_N = 10000
_E = 320000
_D = 128
_H = 128
_C = 64

_NC = 2
_NS = 16
_NW = _NC * _NS
_EPW = _E // _NW
_CHUNK = 125
_NCH = _EPW // _CHUNK
_HCH = _NCH // 2

_NP = 10240
_RPT = _NP // _NS
_OCH = 80
_NOCH = _RPT // _OCH

_BR = 1000
_GB = _N // _BR


def _head_body(pa_ref, pb_ref, w1_ref, b1_ref, wd_ref, bd_ref, o_ref):
    agg = pa_ref[0] + pb_ref[0]
    h = jax.nn.relu(jnp.dot(agg, w1_ref[...],
                            preferred_element_type=jnp.float32) + b1_ref[...])
    o_ref[...] = jnp.dot(h, wd_ref[...],
                         preferred_element_type=jnp.float32) + bd_ref[...]


_sc_mesh = plsc.VectorSubcoreMesh(core_axis_name="c", subcore_axis_name="s")


@functools.partial(
    pl.kernel,
    mesh=_sc_mesh,
    out_type=jax.ShapeDtypeStruct((_NC, _NP, _D), jnp.float32),
    scratch_types=[
        pltpu.VMEM((_HCH, _CHUNK), jnp.int32),
        pltpu.VMEM((_HCH, _CHUNK), jnp.int32),
        pltpu.VMEM((2, _CHUNK, _D), jnp.float32),
        pltpu.VMEM_SHARED((_NP, _D), jnp.float32),
        pltpu.SemaphoreType.DMA,
        pltpu.SemaphoreType.DMA,
    ],
)
def _sc_agg(x_hbm, edges_hbm, out_hbm,
            src_v, dst_v, rows_v, acc_sh, semg, sems):
    cid = lax.axis_index("c")
    sid = lax.axis_index("s")
    wid = sid * _NC + cid

    def _zero_row(r, carry):
        for c in range(_D // 16):
            rows_v[0, r, pl.ds(c * 16, 16)] = jnp.zeros((16,), jnp.float32)
        return carry
    lax.fori_loop(0, _OCH, _zero_row, 0)
    for t in range(_NOCH):
        pltpu.sync_copy(rows_v.at[0, pl.ds(0, _OCH)],
                        acc_sh.at[pl.ds(sid * _RPT + t * _OCH, _OCH)])
    plsc.subcore_barrier()

    for phase in range(2):
        pltpu.sync_copy(edges_hbm.at[0, wid, pl.ds(phase * _HCH, _HCH)],
                        src_v)
        pltpu.sync_copy(edges_hbm.at[1, wid, pl.ds(phase * _HCH, _HCH)],
                        dst_v)
        pltpu.async_copy(x_hbm.at[src_v.at[0]], rows_v.at[0], semg)

        def _edge_chunk(j, carry):
            b = lax.rem(j, 2)
            pltpu.make_async_copy(x_hbm.at[src_v.at[j]],
                                  rows_v.at[b], semg).wait()

            @pl.when(j >= 1)
            def _():
                pltpu.make_async_copy(rows_v.at[1 - b],
                                      acc_sh.at[dst_v.at[j - 1]],
                                      sems).wait()
            pltpu.async_copy(x_hbm.at[src_v.at[j + 1]],
                             rows_v.at[1 - b], semg)
            pltpu.async_copy(rows_v.at[b], acc_sh.at[dst_v.at[j]],
                             sems, add=True)
            return carry
        lax.fori_loop(0, _HCH - 1, _edge_chunk, 0)
        _lb = (_HCH - 1) % 2
        pltpu.make_async_copy(x_hbm.at[src_v.at[_HCH - 1]],
                              rows_v.at[_lb], semg).wait()
        pltpu.make_async_copy(rows_v.at[1 - _lb],
                              acc_sh.at[dst_v.at[_HCH - 2]],
                              sems).wait()
        pltpu.async_copy(rows_v.at[_lb], acc_sh.at[dst_v.at[_HCH - 1]],
                         sems, add=True)
        pltpu.make_async_copy(rows_v.at[_lb],
                              acc_sh.at[dst_v.at[_HCH - 1]],
                              sems).wait()
    plsc.subcore_barrier()

    for t in range(_NOCH):
        base = sid * _RPT + t * _OCH
        pltpu.async_copy(acc_sh.at[pl.ds(base, _OCH)],
                         out_hbm.at[cid, pl.ds(base, _OCH)], semg)
    for t in range(_NOCH):
        pltpu.make_async_copy(acc_sh.at[pl.ds(sid * _RPT, _OCH)],
                              out_hbm.at[cid, pl.ds(sid * _RPT, _OCH)],
                              semg).wait()


def kernel(x, adj, W1, b1, Wd, bd):
    edges = adj.reshape(2, _NW, _NCH, _CHUNK)

    partials = _sc_agg(x, edges)

    out = pl.pallas_call(
        _head_body,
        grid=(_GB,),
        in_specs=[pl.BlockSpec((1, _BR, _D), lambda i: (0, i, 0)),
                  pl.BlockSpec((1, _BR, _D), lambda i: (1, i, 0)),
                  pl.BlockSpec((_D, _H), lambda i: (0, 0)),
                  pl.BlockSpec((1, _H), lambda i: (0, 0)),
                  pl.BlockSpec((_H, _C), lambda i: (0, 0)),
                  pl.BlockSpec((1, _C), lambda i: (0, 0))],
        out_specs=pl.BlockSpec((_BR, _C), lambda i: (i, 0)),
        out_shape=jax.ShapeDtypeStruct((_N, _C), jnp.float32),
    )(partials, partials, W1, b1[None], Wd, bd[None])
    return out
